# baseline (speedup 1.0000x reference)
import functools

import jax
import jax.numpy as jnp
from jax import lax
from jax.experimental import pallas as pl
from jax.experimental.pallas import tpu as pltpu
from jax.experimental.pallas import tpu_sc as plsc

N = 10000
E = 320000
CIN = 128
COUT = 256
NP = 10240
F = 144
NBLK = 2512
EPAD = NBLK * 128
NTILE = 16
TPB = NBLK // NTILE
RB = 2048
GRID = NP // RB
RPT = NP // NTILE


def _h_alpha_body(x_ref, w_ref, aw_ref, hp_ref, hq_ref, alpha_ref, hsum_ref):
    i = pl.program_id(0)
    xb = x_ref[...]
    hb = lax.dot_general(xb, w_ref[...], (((1,), (1,)), ((), ())),
                         preferred_element_type=jnp.float32)
    a = jnp.sum(hb * aw_ref[...], axis=1, keepdims=True)
    s = jnp.sum(hb, axis=1, keepdims=True)
    al = a * s
    alpha_ref[...] = jnp.where(al >= 0, al, 0.2 * al)
    rows = i * RB + lax.broadcasted_iota(jnp.int32, (RB, 1), 0)
    ones = jnp.where(rows < N, 1.0, 0.0).astype(jnp.float32)
    pad = jnp.zeros((RB, F - CIN - 1), jnp.float32)
    hp_ref[...] = jnp.concatenate([hb[:, :CIN], ones, pad], axis=1)
    hq_ref[...] = jnp.concatenate([hb[:, CIN:], ones, pad], axis=1)

    @pl.when(i == 0)
    def _():
        hsum_ref[...] = jnp.zeros_like(hsum_ref)

    hsum_ref[...] += jnp.sum(hb, axis=0, keepdims=True)


_h_alpha = pl.pallas_call(
    _h_alpha_body,
    grid=(GRID,),
    in_specs=[
        pl.BlockSpec((RB, CIN), lambda i: (i, 0)),
        pl.BlockSpec((COUT, CIN), lambda i: (0, 0)),
        pl.BlockSpec((1, COUT), lambda i: (0, 0)),
    ],
    out_specs=[
        pl.BlockSpec((RB, F), lambda i: (i, 0)),
        pl.BlockSpec((RB, F), lambda i: (i, 0)),
        pl.BlockSpec((RB, 1), lambda i: (i, 0)),
        pl.BlockSpec((1, COUT), lambda i: (0, 0)),
    ],
    out_shape=[
        jax.ShapeDtypeStruct((NP, F), jnp.float32),
        jax.ShapeDtypeStruct((NP, F), jnp.float32),
        jax.ShapeDtypeStruct((NP, 1), jnp.float32),
        jax.ShapeDtypeStruct((1, COUT), jnp.float32),
    ],
)


@functools.partial(
    pl.kernel,
    mesh=plsc.VectorSubcoreMesh(core_axis_name="c", subcore_axis_name="s",
                                num_cores=1),
    compiler_params=pltpu.CompilerParams(use_tc_tiling_on_sc=False),
    out_type=jax.ShapeDtypeStruct((NP, F), jnp.float32),
    scratch_types=[
        pltpu.VMEM((128,), jnp.int32),
        pltpu.VMEM((128,), jnp.int32),
        pltpu.VMEM((128, F), jnp.float32),
        pltpu.VMEM((128, F), jnp.float32),
        pltpu.VMEM_SHARED((NP, F), jnp.float32),
        pltpu.SemaphoreType.DMA,
    ],
)
def _seg_sum(hplane, cgf, rsp, zrows, s_out, idx_v, r_v, rows_v, zbuf,
             s_acc, sem):
    s = lax.axis_index("s")
    pltpu.sync_copy(zrows, zbuf)
    for kk in range(RPT // 128):
        pltpu.sync_copy(zbuf, s_acc.at[pl.ds(s * RPT + kk * 128, 128)])
    plsc.subcore_barrier()

    def body(j, carry):
        base = (s * TPB + j) * 128
        pltpu.sync_copy(cgf.at[pl.ds(base, 128)], idx_v)
        pltpu.sync_copy(rsp.at[pl.ds(base, 128)], r_v)
        pltpu.async_copy(hplane.at[idx_v], rows_v, sem).wait()
        pltpu.sync_copy(rows_v, s_acc.at[r_v], add=True)
        return carry

    lax.fori_loop(0, TPB, body, 0)
    plsc.subcore_barrier()
    pltpu.sync_copy(s_acc.at[pl.ds(s * RPT, RPT)],
                    s_out.at[pl.ds(s * RPT, RPT)])


def _final_body(s0_ref, s1_ref, alpha_ref, hsum_ref, lw_ref, lb_ref, o_ref):
    p0 = s0_ref[...]
    p1 = s1_ref[...]
    S = jnp.concatenate([p0[:, :CIN], p1[:, :CIN]], axis=1)
    d = p0[:, CIN:CIN + 1]
    al = alpha_ref[...]
    t = jnp.exp(-jnp.abs(al))
    pos = al >= 0
    coef_s = jnp.where(pos, 1.0 - t, t - 1.0)
    coef_h = jnp.where(pos, t, 1.0)
    num = coef_s * S + coef_h * hsum_ref[...]
    den = coef_s * d + jnp.where(pos, float(N) * t, float(N))
    out = num / den
    y = lax.dot_general(out, lw_ref[...], (((1,), (1,)), ((), ())),
                        preferred_element_type=jnp.float32) + lb_ref[...]
    o_ref[...] = jnp.where(y > 0, y, jnp.exp(jnp.minimum(y, 0.0)) - 1.0)


_final = pl.pallas_call(
    _final_body,
    grid=(GRID,),
    in_specs=[
        pl.BlockSpec((RB, F), lambda i: (i, 0)),
        pl.BlockSpec((RB, F), lambda i: (i, 0)),
        pl.BlockSpec((RB, 1), lambda i: (i, 0)),
        pl.BlockSpec((1, COUT), lambda i: (0, 0)),
        pl.BlockSpec((COUT, COUT), lambda i: (0, 0)),
        pl.BlockSpec((1, COUT), lambda i: (0, 0)),
    ],
    out_specs=pl.BlockSpec((RB, COUT), lambda i: (i, 0)),
    out_shape=jax.ShapeDtypeStruct((NP, COUT), jnp.float32),
)


def kernel(x, edge_index, W, att_w, lin_w, lin_b):
    row = edge_index[0].astype(jnp.int32)
    col = edge_index[1].astype(jnp.int32)
    key = row * N + col
    sk = jnp.sort(key)
    uniq = jnp.concatenate([jnp.ones((1,), jnp.bool_), sk[1:] != sk[:-1]])
    r_s = sk // N
    c_s = sk - r_s * N
    spread = jnp.arange(E, dtype=jnp.int32) % 32
    cg = jnp.where(uniq, c_s, N + spread)
    padspread = jnp.arange(EPAD - E, dtype=jnp.int32) % 32
    cgf = jnp.concatenate([cg, N + padspread])
    rsp = jnp.concatenate([r_s, N + 100 + padspread]).astype(jnp.int32)

    x_pad = jnp.zeros((NP, CIN), jnp.float32).at[:N].set(x)
    zrows = jnp.zeros((128, F), jnp.float32)

    h0, h1, alpha, hsum = _h_alpha(x_pad, W, att_w)
    s0 = _seg_sum(h0, cgf, rsp, zrows)
    s1 = _seg_sum(h1, cgf, rsp, zrows)
    out = _final(s0, s1, alpha, hsum, lin_w,
                 lin_b.reshape(1, COUT))
    return out[:N]


# speedup vs baseline: 1.4782x; 1.4782x over previous
import functools

import jax
import jax.numpy as jnp
from jax import lax
from jax.experimental import pallas as pl
from jax.experimental.pallas import tpu as pltpu
from jax.experimental.pallas import tpu_sc as plsc

N = 10000
E = 320000
CIN = 128
COUT = 256
NP = 10240
F = 144
NBLK = 2512
EPAD = NBLK * 128
NTILE = 16
TPB = NBLK // NTILE
RB = 2048
GRID = NP // RB
RPT = NP // NTILE


def _h_alpha_body(x_ref, w_ref, aw_ref, hp_ref, hq_ref, alpha_ref, hsum_ref):
    i = pl.program_id(0)
    xb = x_ref[...]
    hb = lax.dot_general(xb, w_ref[...], (((1,), (1,)), ((), ())),
                         preferred_element_type=jnp.float32)
    a = jnp.sum(hb * aw_ref[...], axis=1, keepdims=True)
    s = jnp.sum(hb, axis=1, keepdims=True)
    al = a * s
    alpha_ref[...] = jnp.where(al >= 0, al, 0.2 * al)
    rows = i * RB + lax.broadcasted_iota(jnp.int32, (RB, 1), 0)
    ones = jnp.where(rows < N, 1.0, 0.0).astype(jnp.float32)
    pad = jnp.zeros((RB, F - CIN - 1), jnp.float32)
    hp_ref[...] = jnp.concatenate([hb[:, :CIN], ones, pad], axis=1)
    hq_ref[...] = jnp.concatenate([hb[:, CIN:], ones, pad], axis=1)

    @pl.when(i == 0)
    def _():
        hsum_ref[...] = jnp.zeros_like(hsum_ref)

    hsum_ref[...] += jnp.sum(hb, axis=0, keepdims=True)


_h_alpha = pl.pallas_call(
    _h_alpha_body,
    grid=(GRID,),
    in_specs=[
        pl.BlockSpec((RB, CIN), lambda i: (i, 0)),
        pl.BlockSpec((COUT, CIN), lambda i: (0, 0)),
        pl.BlockSpec((1, COUT), lambda i: (0, 0)),
    ],
    out_specs=[
        pl.BlockSpec((RB, F), lambda i: (i, 0)),
        pl.BlockSpec((RB, F), lambda i: (i, 0)),
        pl.BlockSpec((RB, 1), lambda i: (i, 0)),
        pl.BlockSpec((1, COUT), lambda i: (0, 0)),
    ],
    out_shape=[
        jax.ShapeDtypeStruct((NP, F), jnp.float32),
        jax.ShapeDtypeStruct((NP, F), jnp.float32),
        jax.ShapeDtypeStruct((NP, 1), jnp.float32),
        jax.ShapeDtypeStruct((1, COUT), jnp.float32),
    ],
)


@functools.partial(
    pl.kernel,
    mesh=plsc.VectorSubcoreMesh(core_axis_name="c", subcore_axis_name="s"),
    compiler_params=pltpu.CompilerParams(use_tc_tiling_on_sc=False),
    out_type=[jax.ShapeDtypeStruct((NP, F), jnp.float32),
              jax.ShapeDtypeStruct((NP, F), jnp.float32)],
    scratch_types=[
        pltpu.VMEM((128,), jnp.int32),
        pltpu.VMEM((128,), jnp.int32),
        pltpu.VMEM((128, F), jnp.float32),
        pltpu.VMEM((128, F), jnp.float32),
        pltpu.VMEM_SHARED((NP, F), jnp.float32),
        pltpu.SemaphoreType.DMA,
    ],
)
def _seg_sum(h0, h1, cgf, rsp, zrows, s_out0, s_out1, idx_v, r_v, rows_v, zbuf,
             s_acc, sem):
    c = lax.axis_index("c")
    s = lax.axis_index("s")
    pltpu.sync_copy(zrows, zbuf)
    for kk in range(RPT // 128):
        pltpu.sync_copy(zbuf, s_acc.at[pl.ds(s * RPT + kk * 128, 128)])
    plsc.subcore_barrier()

    def body0(j, carry):
        base = (s * TPB + j) * 128
        pltpu.sync_copy(cgf.at[pl.ds(base, 128)], idx_v)
        pltpu.sync_copy(rsp.at[pl.ds(base, 128)], r_v)
        pltpu.async_copy(h0.at[idx_v], rows_v, sem).wait()
        pltpu.sync_copy(rows_v, s_acc.at[r_v], add=True)
        return carry

    def body1(j, carry):
        base = (s * TPB + j) * 128
        pltpu.sync_copy(cgf.at[pl.ds(base, 128)], idx_v)
        pltpu.sync_copy(rsp.at[pl.ds(base, 128)], r_v)
        pltpu.async_copy(h1.at[idx_v], rows_v, sem).wait()
        pltpu.sync_copy(rows_v, s_acc.at[r_v], add=True)
        return carry

    @pl.when(c == 0)
    def _():
        lax.fori_loop(0, TPB, body0, 0)

    @pl.when(c == 1)
    def _():
        lax.fori_loop(0, TPB, body1, 0)

    plsc.subcore_barrier()

    @pl.when(c == 0)
    def _():
        pltpu.sync_copy(s_acc.at[pl.ds(s * RPT, RPT)],
                        s_out0.at[pl.ds(s * RPT, RPT)])

    @pl.when(c == 1)
    def _():
        pltpu.sync_copy(s_acc.at[pl.ds(s * RPT, RPT)],
                        s_out1.at[pl.ds(s * RPT, RPT)])


def _final_body(s0_ref, s1_ref, alpha_ref, hsum_ref, lw_ref, lb_ref, o_ref):
    p0 = s0_ref[...]
    p1 = s1_ref[...]
    S = jnp.concatenate([p0[:, :CIN], p1[:, :CIN]], axis=1)
    d = p0[:, CIN:CIN + 1]
    al = alpha_ref[...]
    t = jnp.exp(-jnp.abs(al))
    pos = al >= 0
    coef_s = jnp.where(pos, 1.0 - t, t - 1.0)
    coef_h = jnp.where(pos, t, 1.0)
    num = coef_s * S + coef_h * hsum_ref[...]
    den = coef_s * d + jnp.where(pos, float(N) * t, float(N))
    out = num / den
    y = lax.dot_general(out, lw_ref[...], (((1,), (1,)), ((), ())),
                        preferred_element_type=jnp.float32) + lb_ref[...]
    o_ref[...] = jnp.where(y > 0, y, jnp.exp(jnp.minimum(y, 0.0)) - 1.0)


_final = pl.pallas_call(
    _final_body,
    grid=(GRID,),
    in_specs=[
        pl.BlockSpec((RB, F), lambda i: (i, 0)),
        pl.BlockSpec((RB, F), lambda i: (i, 0)),
        pl.BlockSpec((RB, 1), lambda i: (i, 0)),
        pl.BlockSpec((1, COUT), lambda i: (0, 0)),
        pl.BlockSpec((COUT, COUT), lambda i: (0, 0)),
        pl.BlockSpec((1, COUT), lambda i: (0, 0)),
    ],
    out_specs=pl.BlockSpec((RB, COUT), lambda i: (i, 0)),
    out_shape=jax.ShapeDtypeStruct((NP, COUT), jnp.float32),
)


def kernel(x, edge_index, W, att_w, lin_w, lin_b):
    row = edge_index[0].astype(jnp.int32)
    col = edge_index[1].astype(jnp.int32)
    key = row * N + col
    sk = jnp.sort(key)
    uniq = jnp.concatenate([jnp.ones((1,), jnp.bool_), sk[1:] != sk[:-1]])
    r_s = sk // N
    c_s = sk - r_s * N
    spread = jnp.arange(E, dtype=jnp.int32) % 32
    cg = jnp.where(uniq, c_s, N + spread)
    padspread = jnp.arange(EPAD - E, dtype=jnp.int32) % 32
    cgf = jnp.concatenate([cg, N + padspread])
    rsp = jnp.concatenate([r_s, N + 100 + padspread]).astype(jnp.int32)
    cgf = cgf.reshape(128, NBLK).T.reshape(-1)
    rsp = rsp.reshape(128, NBLK).T.reshape(-1)

    x_pad = jnp.zeros((NP, CIN), jnp.float32).at[:N].set(x)
    zrows = jnp.zeros((128, F), jnp.float32)

    h0, h1, alpha, hsum = _h_alpha(x_pad, W, att_w)
    s0, s1 = _seg_sum(h0, h1, cgf, rsp, zrows)
    out = _final(s0, s1, alpha, hsum, lin_w,
                 lin_b.reshape(1, COUT))
    return out[:N]


# speedup vs baseline: 1.5863x; 1.0731x over previous
import functools

import jax
import jax.numpy as jnp
from jax import lax
from jax.experimental import pallas as pl
from jax.experimental.pallas import tpu as pltpu
from jax.experimental.pallas import tpu_sc as plsc

N = 10000
E = 320000
CIN = 128
COUT = 256
NP = 10240
FC = 64
F = 80
NBLK = 2512
EPAD = NBLK * 128
NTILE = 16
TPB = NBLK // NTILE
RB = 2048
GRID = NP // RB
RPT = NP // NTILE


def _h_alpha_body(x_ref, w_ref, aw_ref, h0_ref, h1_ref, h2_ref, h3_ref,
                  alpha_ref, hsum_ref):
    i = pl.program_id(0)
    xb = x_ref[...]
    hb = lax.dot_general(xb, w_ref[...], (((1,), (1,)), ((), ())),
                         preferred_element_type=jnp.float32)
    a = jnp.sum(hb * aw_ref[...], axis=1, keepdims=True)
    s = jnp.sum(hb, axis=1, keepdims=True)
    al = a * s
    alpha_ref[...] = jnp.where(al >= 0, al, 0.2 * al)
    rows = i * RB + lax.broadcasted_iota(jnp.int32, (RB, 1), 0)
    ones = jnp.where(rows < N, 1.0, 0.0).astype(jnp.float32)
    pad = jnp.zeros((RB, F - FC - 1), jnp.float32)
    for k, ref in enumerate((h0_ref, h1_ref, h2_ref, h3_ref)):
        ref[...] = jnp.concatenate(
            [hb[:, k * FC:(k + 1) * FC], ones, pad], axis=1)

    @pl.when(i == 0)
    def _():
        hsum_ref[...] = jnp.zeros_like(hsum_ref)

    hsum_ref[...] += jnp.sum(hb, axis=0, keepdims=True)


_h_alpha = pl.pallas_call(
    _h_alpha_body,
    grid=(GRID,),
    in_specs=[
        pl.BlockSpec((RB, CIN), lambda i: (i, 0)),
        pl.BlockSpec((COUT, CIN), lambda i: (0, 0)),
        pl.BlockSpec((1, COUT), lambda i: (0, 0)),
    ],
    out_specs=[pl.BlockSpec((RB, F), lambda i: (i, 0))] * 4 + [
        pl.BlockSpec((RB, 1), lambda i: (i, 0)),
        pl.BlockSpec((1, COUT), lambda i: (0, 0)),
    ],
    out_shape=[jax.ShapeDtypeStruct((NP, F), jnp.float32)] * 4 + [
        jax.ShapeDtypeStruct((NP, 1), jnp.float32),
        jax.ShapeDtypeStruct((1, COUT), jnp.float32),
    ],
)


@functools.partial(
    pl.kernel,
    mesh=plsc.VectorSubcoreMesh(core_axis_name="c", subcore_axis_name="s"),
    compiler_params=pltpu.CompilerParams(use_tc_tiling_on_sc=False),
    out_type=[pltpu.HBM((NP, F), jnp.float32),
              pltpu.HBM((NP, F), jnp.float32)],
    scratch_types=[
        pltpu.VMEM((TPB * 128,), jnp.int32),
        pltpu.VMEM((TPB * 128,), jnp.int32),
        pltpu.VMEM((128, F), jnp.float32),
        pltpu.VMEM((128, F), jnp.float32),
        pltpu.VMEM((128, F), jnp.float32),
        pltpu.VMEM_SHARED((NP, F), jnp.float32),
        pltpu.SemaphoreType.DMA,
        pltpu.SemaphoreType.DMA,
    ],
)
def _seg_sum(ha, hb, cgf, rsp3, zrows, s_out0, s_out1, idx_all, r3, rows_a,
             rows_b, zbuf, s_acc, sem_a, sem_b):
    c = lax.axis_index("c")
    s = lax.axis_index("s")
    pltpu.sync_copy(zrows, zbuf)
    for kk in range(RPT // 128):
        pltpu.sync_copy(zbuf, s_acc.at[pl.ds(s * RPT + kk * 128, 128)])
    pltpu.sync_copy(cgf.at[pl.ds(s * TPB * 128, TPB * 128)], idx_all)
    pltpu.sync_copy(rsp3.at[pl.ds(s * TPB * 128, TPB * 128)], r3)
    plsc.subcore_barrier()

    def run(hsel):
        def start(j, buf, sem):
            pltpu.async_copy(hsel.at[idx_all.at[pl.ds(j * 128, 128)]],
                             buf, sem)

        def wait(buf, sem):
            pltpu.make_async_copy(zrows, buf, sem).wait()

        def scatter(buf, j):
            pltpu.sync_copy(buf, s_acc.at[r3.at[pl.ds(j * 128, 128)]],
                            add=True)

        start(0, rows_a, sem_a)
        def body(i, carry):
            j = 2 * i
            wait(rows_a, sem_a)
            start(j + 1, rows_b, sem_b)
            scatter(rows_a, j)
            wait(rows_b, sem_b)
            start(j + 2, rows_a, sem_a)
            scatter(rows_b, j + 1)
            return carry

        lax.fori_loop(0, (TPB - 1) // 2, body, 0)
        wait(rows_a, sem_a)
        scatter(rows_a, TPB - 1)

    @pl.when(c == 0)
    def _():
        run(ha)

    @pl.when(c == 1)
    def _():
        run(hb)

    plsc.subcore_barrier()

    @pl.when(c == 0)
    def _():
        pltpu.sync_copy(s_acc.at[pl.ds(s * RPT, RPT)],
                        s_out0.at[pl.ds(s * RPT, RPT)])

    @pl.when(c == 1)
    def _():
        pltpu.sync_copy(s_acc.at[pl.ds(s * RPT, RPT)],
                        s_out1.at[pl.ds(s * RPT, RPT)])


def _final_body(s0_ref, s1_ref, s2_ref, s3_ref, alpha_ref, hsum_ref, lw_ref,
                lb_ref, o_ref):
    p0 = s0_ref[...]
    S = jnp.concatenate(
        [p0[:, :FC], s1_ref[:, :FC], s2_ref[:, :FC], s3_ref[:, :FC]], axis=1)
    d = p0[:, FC:FC + 1]
    al = alpha_ref[...]
    t = jnp.exp(-jnp.abs(al))
    pos = al >= 0
    coef_s = jnp.where(pos, 1.0 - t, t - 1.0)
    coef_h = jnp.where(pos, t, 1.0)
    num = coef_s * S + coef_h * hsum_ref[...]
    den = coef_s * d + jnp.where(pos, float(N) * t, float(N))
    out = num / den
    y = lax.dot_general(out, lw_ref[...], (((1,), (1,)), ((), ())),
                        preferred_element_type=jnp.float32) + lb_ref[...]
    o_ref[...] = jnp.where(y > 0, y, jnp.exp(jnp.minimum(y, 0.0)) - 1.0)


_final = pl.pallas_call(
    _final_body,
    grid=(GRID,),
    in_specs=[pl.BlockSpec((RB, F), lambda i: (i, 0))] * 4 + [
        pl.BlockSpec((RB, 1), lambda i: (i, 0)),
        pl.BlockSpec((1, COUT), lambda i: (0, 0)),
        pl.BlockSpec((COUT, COUT), lambda i: (0, 0)),
        pl.BlockSpec((1, COUT), lambda i: (0, 0)),
    ],
    out_specs=pl.BlockSpec((RB, COUT), lambda i: (i, 0)),
    out_shape=jax.ShapeDtypeStruct((NP, COUT), jnp.float32),
)


def kernel(x, edge_index, W, att_w, lin_w, lin_b):
    row = edge_index[0].astype(jnp.int32)
    col = edge_index[1].astype(jnp.int32)
    key = row * N + col
    sk = jnp.sort(key)
    uniq = jnp.concatenate([jnp.ones((1,), jnp.bool_), sk[1:] != sk[:-1]])
    r_s = sk // N
    c_s = sk - r_s * N
    spread = jnp.arange(E, dtype=jnp.int32) % 32
    cg = jnp.where(uniq, c_s, N + spread)
    padspread = jnp.arange(EPAD - E, dtype=jnp.int32) % 32
    cgf = jnp.concatenate([cg, N + padspread])
    rsp = jnp.concatenate([r_s, N + 100 + padspread]).astype(jnp.int32)
    cgf = cgf.reshape(128, NBLK).T.reshape(-1)
    rsp = rsp.reshape(128, NBLK).T.reshape(-1)

    x_pad = jnp.zeros((NP, CIN), jnp.float32).at[:N].set(x)
    zrows = jnp.zeros((128, F), jnp.float32)

    h0, h1, h2, h3, alpha, hsum = _h_alpha(x_pad, W, att_w)
    hbm = lambda a: pltpu.with_memory_space_constraint(
        a, pltpu.MemorySpace.HBM)
    cgf, rsp = hbm(cgf), hbm(rsp)

    def body(carry, planes):
        ha, hb = planes
        sa, sb = _seg_sum(hbm(ha), hbm(hb), cgf, rsp, zrows)
        return carry, (sa, sb)

    _, (sa, sb) = lax.scan(
        body, 0, (jnp.stack([h0, h2]), jnp.stack([h1, h3])))
    out = _final(sa[0], sb[0], sa[1], sb[1], alpha, hsum, lin_w,
                 lin_b.reshape(1, COUT))
    return out[:N]
